# baseline (speedup 1.0000x reference)
import functools

import jax
import jax.numpy as jnp
from jax import lax
from jax.experimental import pallas as pl
from jax.experimental.pallas import tpu as pltpu
from jax.experimental.pallas import tpu_sc as plsc

N = 320000
D = 128
S = 10000
NW = 32
RPW = N // NW
CH = 80
NCH = RPW // CH
NG = CH // 16
NB_PAD = 96
NBATCH = NB_PAD // 16
RING = 8
F32 = jnp.float32
I32 = jnp.int32

_mesh = plsc.VectorSubcoreMesh(core_axis_name="c", subcore_axis_name="s")
_params = pltpu.CompilerParams(needs_layout_passes=False)


def _sread(ref, i):
    return ref[pl.ds(i, 16)][0]


def _scan_chunk(ib, base, prev0, pos, ids2=None):
    lanes = lax.iota(I32, 16)
    nb = jnp.int32(0)
    prev = prev0
    for g in range(NG):
        v = ib[pl.ds(base + g * 16, 16)]
        shifted = jnp.take(v, jnp.maximum(lanes - 1, 0))
        shifted = jnp.where(lanes == 0, prev, shifted)
        m = v != shifted
        cs = jnp.cumsum(m.astype(I32))
        tgt = nb + cs - 1
        plsc.store_scatter(pos, [tgt], lanes + g * 16, mask=m)
        if ids2 is not None:
            plsc.store_scatter(ids2, [tgt // 16, tgt % 16], v, mask=m)
        nb = nb + cs[15]
        prev = v[15]
    cur = pos[pl.ds(nb, 16)]
    pos[pl.ds(nb, 16)] = jnp.where(lanes == 0, CH, cur)
    if ids2 is not None:
        for b in range(NBATCH):
            lanepos = lanes + b * 16
            curi = ids2[b]
            ids2[b] = jnp.where(lanepos < nb, curi, 0)
    return nb


def _p1_body(x_hbm, idx_hbm, t_hbm, w_hbm, bc_hbm, ib, xb, pos, prevb, tb,
             wb, ring, xsem, fsem):
    wid = lax.axis_index("s") * 2 + lax.axis_index("c")
    r0 = pl.multiple_of(wid * RPW, 8)

    pltpu.sync_copy(idx_hbm.at[pl.ds(r0, RPW)], ib.at[pl.ds(0, RPW)])
    pltpu.sync_copy(t_hbm, tb)
    pltpu.sync_copy(w_hbm, wb)
    for j in range(8):
        tv = tb[pl.ds(j * 16, 16)]
        tb[pl.ds(j * 16, 16)] = jnp.clip(tv, 0.0, 1.0)

    @pl.when(wid > 0)
    def _():
        pltpu.sync_copy(idx_hbm.at[pl.ds(pl.multiple_of(r0 - 8, 8), 16)], prevb)

    tile_prev = jnp.where(wid > 0, prevb[pl.ds(0, 16)][7], jnp.int32(-1))
    wv = wb[pl.ds(0, 16)]
    w0, w1, w2, w4 = wv[0], wv[1], wv[2], wv[4]

    pinf = jnp.full((16,), 3.4e38, F32)
    ninf = jnp.full((16,), -3.4e38, F32)
    zero = jnp.zeros((16,), F32)

    def flush(cur_seg, cnt, mn, mx, sm, dma_cnt):
        @pl.when(cur_seg != tile_prev)
        def _():
            @pl.when(dma_cnt >= RING)
            def _():
                pltpu.make_async_copy(ring.at[0], bc_hbm.at[pl.ds(0, 2 * D)],
                                      fsem).wait()

            slot = dma_cnt & (RING - 1)
            cntf = cnt.astype(F32)
            for j in range(8):
                ttj = tb[pl.ds(j * 16, 16)]
                ring[slot, pl.ds(j * 16, 16)] = (
                    ttj * mx[j] + (1.0 - ttj) * mn[j])
                ring[slot, pl.ds(128 + j * 16, 16)] = (
                    w0 * cntf + w1 * mn[j] + w2 * mx[j] + w4 * sm[j])
            pltpu.async_copy(
                ring.at[slot],
                bc_hbm.at[pl.ds(pl.multiple_of(cur_seg * (2 * D), 8),
                               2 * D)], fsem)

        return jnp.where(cur_seg != tile_prev, dma_cnt + 1, dma_cnt)

    def accum_rows(xbase, lo, hi, cnt, mn, mx, sm):
        def row(i, carry):
            cnt, mn, mx, sm = carry
            mn2, mx2, sm2 = [], [], []
            for j in range(8):
                ld = xb[xbase + i, pl.ds(j * 16, 16)]
                mn2.append(jnp.minimum(mn[j], ld))
                mx2.append(jnp.maximum(mx[j], ld))
                sm2.append(sm[j] + ld)
            return cnt + 1, tuple(mn2), tuple(mx2), tuple(sm2)

        return lax.fori_loop(lo, hi, row, (cnt, mn, mx, sm))

    pltpu.async_copy(x_hbm.at[pl.ds(r0, CH)], xb.at[pl.ds(0, CH)],
                     xsem)

    def chunk(k, carry):
        cur_seg, cnt, dma_cnt, mn, mx, sm = carry
        xbase = (k & 1) * CH
        pltpu.make_async_copy(x_hbm.at[pl.ds(0, CH)], xb.at[pl.ds(0, CH)],
                              xsem).wait()

        @pl.when(k + 1 < NCH)
        def _():
            pltpu.async_copy(x_hbm.at[pl.ds(pl.multiple_of(r0 + (k + 1) * CH, 8),
                                            CH)],
                             xb.at[pl.ds(((k + 1) & 1) * CH, CH)], xsem)

        prev0 = jnp.where(k > 0, _sread(ib, jnp.maximum(k * CH - 1, 0)),
                          tile_prev)
        nb = _scan_chunk(ib, k * CH, prev0, pos)
        p0 = jnp.where(nb > 0, _sread(pos, 0), CH)
        cnt, mn, mx, sm = accum_rows(xbase, 0, p0, cnt, mn, mx, sm)

        def seg(j, c):
            cur_seg, cnt, dma_cnt, mn, mx, sm = c
            p_lo = _sread(pos, j)
            p_hi = _sread(pos, j + 1)
            dma_cnt = flush(cur_seg, cnt, mn, mx, sm, dma_cnt)
            new_seg = _sread(ib, k * CH + p_lo)
            cnt, mn, mx, sm = accum_rows(
                xbase, p_lo, p_hi, jnp.int32(0),
                (pinf,) * 8, (ninf,) * 8, (zero,) * 8)
            return new_seg, cnt, dma_cnt, mn, mx, sm

        return lax.fori_loop(0, nb, seg,
                             (cur_seg, cnt, dma_cnt, mn, mx, sm))

    init = (tile_prev, jnp.int32(0), jnp.int32(0),
            (pinf,) * 8, (ninf,) * 8, (zero,) * 8)
    cur_seg, cnt, dma_cnt, mn, mx, sm = lax.fori_loop(0, NCH, chunk, init)

    first_ec = pl.multiple_of(r0 + RPW, 8)

    @pl.when(first_ec < N)
    def _():
        pltpu.sync_copy(idx_hbm.at[pl.ds(pl.multiple_of(first_ec, 8), 16)], prevb)

    nxt_id = jnp.where(first_ec < N, prevb[pl.ds(0, 16)][0], jnp.int32(-1))
    cont0 = (nxt_id == cur_seg) & (first_ec < N)

    def ext_cond(c):
        return c[0]

    def ext_body(c):
        _, ec0, cnt, mn, mx, sm = c
        pltpu.sync_copy(idx_hbm.at[pl.ds(pl.multiple_of(ec0, 8), CH)],
                        ib.at[pl.ds(0, CH)])
        pltpu.sync_copy(x_hbm.at[pl.ds(pl.multiple_of(ec0, 8), CH)],
                        xb.at[pl.ds(0, CH)])
        nb = _scan_chunk(ib, 0, cur_seg, pos)
        fp = jnp.where(nb > 0, _sread(pos, 0), CH)
        cnt, mn, mx, sm = accum_rows(0, 0, fp, cnt, mn, mx, sm)
        cont = (nb == 0) & (ec0 + CH < N)
        return cont, pl.multiple_of(ec0 + CH, 8), cnt, mn, mx, sm

    _, _, cnt, mn, mx, sm = lax.while_loop(
        ext_cond, ext_body, (cont0, first_ec, cnt, mn, mx, sm))

    dma_cnt = flush(cur_seg, cnt, mn, mx, sm, dma_cnt)

    def drain(i, _):
        @pl.when(i < jnp.minimum(dma_cnt, RING))
        def _():
            pltpu.make_async_copy(ring.at[0], bc_hbm.at[pl.ds(0, 2 * D)],
                                      fsem).wait()

        return 0

    lax.fori_loop(0, RING, drain, 0)


def _p2_body(x_hbm, idx_hbm, w_hbm, bc_hbm, out_hbm, ib, xb, pos2, ids2,
             slab, prevb, wb, ring, zbuf, xsem, gsem, fsem):
    wid = lax.axis_index("s") * 2 + lax.axis_index("c")
    r0 = pl.multiple_of(wid * RPW, 8)

    pltpu.sync_copy(idx_hbm.at[pl.ds(r0, RPW)], ib.at[pl.ds(0, RPW)])
    pltpu.sync_copy(w_hbm, wb)

    @pl.when(wid > 0)
    def _():
        pltpu.sync_copy(idx_hbm.at[pl.ds(pl.multiple_of(r0 - 8, 8), 16)], prevb)

    tile_prev = jnp.where(wid > 0, prevb[pl.ds(0, 16)][7], jnp.int32(-1))
    w3 = wb[pl.ds(0, 16)][3]
    zero = jnp.zeros((16,), F32)
    for r in range(16 * 8):
        zbuf[pl.ds(r * 16, 16)] = zero

    def zfill(lo, hi):
        nfull = jnp.maximum((hi - lo) // 16, 0)

        def f16(i, _):
            pltpu.sync_copy(
                zbuf,
                out_hbm.at[pl.ds(pl.multiple_of((lo + i * 16) * D, 8),
                                 16 * D)])
            return 0

        lax.fori_loop(0, nfull, f16, 0)

        def f1(g, _):
            pltpu.sync_copy(
                zbuf.at[pl.ds(0, D)],
                out_hbm.at[pl.ds(pl.multiple_of(g * D, 8), D)])
            return 0

        lax.fori_loop(lo + nfull * 16, hi, f1, 0)

    def flush(cur_seg, new_seg, acc, cvec, dma_cnt):
        zfill(cur_seg + 1, new_seg)

        @pl.when(cur_seg != tile_prev)
        def _():
            @pl.when(dma_cnt >= RING)
            def _():
                pltpu.make_async_copy(ring.at[0],
                                      out_hbm.at[pl.ds(0, D)],
                                      fsem).wait()

            slot = dma_cnt & (RING - 1)
            for j in range(8):
                ring[slot, pl.ds(j * 16, 16)] = cvec[j] + w3 * acc[j]
            pltpu.async_copy(
                ring.at[slot],
                out_hbm.at[pl.ds(pl.multiple_of(cur_seg * D, 8), D)], fsem)

        return jnp.where(cur_seg != tile_prev, dma_cnt + 1, dma_cnt)

    def relu_rows(xbase, lo, hi, acc, bias):
        def row(i, a):
            a2 = []
            for j in range(8):
                ld = xb[xbase + i, pl.ds(j * 16, 16)]
                a2.append(a[j] + jnp.maximum(ld - bias[j], 0.0))
            return tuple(a2)

        return lax.fori_loop(lo, hi, row, acc)

    def scan_and_gather(kk, par):
        prev0 = _sread(ib, jnp.maximum(kk * CH - 1, 0))
        prev0 = jnp.where(kk > 0, prev0, tile_prev)
        nb = _scan_chunk(ib, kk * CH, prev0, pos2.at[par], ids2.at[par])
        for b in range(NBATCH):
            pltpu.async_copy(bc_hbm.at[ids2.at[par, b]],
                             slab.at[pl.ds(par * NB_PAD + b * 16, 16)],
                             gsem)
        return nb

    pltpu.async_copy(x_hbm.at[pl.ds(r0, CH)], xb.at[pl.ds(0, CH)],
                     xsem)
    nb0 = scan_and_gather(jnp.int32(0), 0)

    def chunk(k, carry):
        cur_seg, dma_cnt, nb, acc, bias, cvec = carry
        par = k & 1
        xbase = par * CH
        pltpu.make_async_copy(x_hbm.at[pl.ds(0, CH)], xb.at[pl.ds(0, CH)],
                              xsem).wait()
        for _b in range(NBATCH):
            pltpu.make_async_copy(bc_hbm.at[ids2.at[0, 0]],
                                  slab.at[pl.ds(0, 16)], gsem).wait()

        @pl.when(k + 1 < NCH)
        def _():
            pltpu.async_copy(x_hbm.at[pl.ds(pl.multiple_of(r0 + (k + 1) * CH, 8),
                                            CH)],
                             xb.at[pl.ds(((k + 1) & 1) * CH, CH)], xsem)

        nb_next = scan_and_gather(jnp.minimum(k + 1, NCH - 1), (k + 1) & 1)

        p0 = jnp.where(nb > 0, _sread(pos2.at[par], 0), CH)
        acc = relu_rows(xbase, 0, p0, acc, bias)

        def seg(j, c):
            cur_seg, dma_cnt, acc, bias, cvec = c
            p_lo = _sread(pos2.at[par], j)
            p_hi = _sread(pos2.at[par], j + 1)
            new_seg = _sread(ib, k * CH + p_lo)
            dma_cnt = flush(cur_seg, new_seg, acc, cvec, dma_cnt)
            srow = par * NB_PAD + j
            nbias = tuple(
                slab[srow, pl.ds(jj * 16, 16)] for jj in range(8))
            ncvec = tuple(
                slab[srow, pl.ds(128 + jj * 16, 16)] for jj in range(8))
            acc = relu_rows(xbase, p_lo, p_hi, (zero,) * 8, nbias)
            return new_seg, dma_cnt, acc, nbias, ncvec

        cur_seg, dma_cnt, acc, bias, cvec = lax.fori_loop(
            0, nb, seg, (cur_seg, dma_cnt, acc, bias, cvec))
        return cur_seg, dma_cnt, nb_next, acc, bias, cvec

    init = (tile_prev, jnp.int32(0), nb0, (zero,) * 8, (zero,) * 8,
            (zero,) * 8)
    cur_seg, dma_cnt, _, acc, bias, cvec = lax.fori_loop(
        0, NCH, chunk, init)

    for _b in range(NBATCH):
        pltpu.make_async_copy(bc_hbm.at[ids2.at[0, 0]],
                              slab.at[pl.ds(0, 16)], gsem).wait()

    first_ec = pl.multiple_of(r0 + RPW, 8)

    @pl.when(first_ec < N)
    def _():
        pltpu.sync_copy(idx_hbm.at[pl.ds(pl.multiple_of(first_ec, 8), 16)], prevb)

    nxt_id = jnp.where(first_ec < N, prevb[pl.ds(0, 16)][0], jnp.int32(-1))
    cont0 = (nxt_id == cur_seg) & (first_ec < N)

    def ext_cond(c):
        return c[0]

    def ext_body(c):
        _, ec0, acc = c
        pltpu.sync_copy(idx_hbm.at[pl.ds(pl.multiple_of(ec0, 8), CH)],
                        ib.at[pl.ds(0, CH)])
        pltpu.sync_copy(x_hbm.at[pl.ds(pl.multiple_of(ec0, 8), CH)],
                        xb.at[pl.ds(0, CH)])
        nb = _scan_chunk(ib, 0, cur_seg, pos2.at[0])
        fp = jnp.where(nb > 0, _sread(pos2.at[0], 0), CH)
        acc = relu_rows(0, 0, fp, acc, bias)
        cont = (nb == 0) & (ec0 + CH < N)
        return cont, pl.multiple_of(ec0 + CH, 8), acc

    _, _, acc = lax.while_loop(ext_cond, ext_body, (cont0, first_ec, acc))

    dma_cnt = flush(cur_seg, jnp.where(wid == NW - 1, S, cur_seg + 1),
                    acc, cvec, dma_cnt)

    def drain(i, _):
        @pl.when(i < jnp.minimum(dma_cnt, RING))
        def _():
            pltpu.make_async_copy(ring.at[0], out_hbm.at[pl.ds(0, D)],
                                      fsem).wait()

        return 0

    lax.fori_loop(0, RING, drain, 0)


_pass1 = functools.partial(
    pl.kernel,
    out_type=jax.ShapeDtypeStruct((S * 2 * D,), F32),
    mesh=_mesh,
    compiler_params=_params,
    scratch_types=[
        pltpu.VMEM((RPW + 16,), I32),
        pltpu.VMEM((2 * CH, D), F32),
        pltpu.VMEM((NB_PAD + 16,), I32),
        pltpu.VMEM((16,), I32),
        pltpu.VMEM((D,), F32),
        pltpu.VMEM((16,), F32),
        pltpu.VMEM((RING, 2 * D), F32),
        pltpu.SemaphoreType.DMA,
        pltpu.SemaphoreType.DMA,
    ],
)(_p1_body)

_pass2 = functools.partial(
    pl.kernel,
    out_type=jax.ShapeDtypeStruct((S * D,), F32),
    mesh=_mesh,
    compiler_params=_params,
    scratch_types=[
        pltpu.VMEM((RPW + 16,), I32),
        pltpu.VMEM((2 * CH, D), F32),
        pltpu.VMEM((2, NB_PAD + 16), I32),
        pltpu.VMEM((2, NBATCH, 16), I32),
        pltpu.VMEM((2 * NB_PAD, 2 * D), F32),
        pltpu.VMEM((16,), I32),
        pltpu.VMEM((16,), F32),
        pltpu.VMEM((RING, D), F32),
        pltpu.VMEM((16 * D,), F32),
        pltpu.SemaphoreType.DMA,
        pltpu.SemaphoreType.DMA,
        pltpu.SemaphoreType.DMA,
    ],
)(_p2_body)


def kernel(x, batch_idx, max_index, t, W):
    assert x.shape == (N, D)
    idx = batch_idx.astype(I32)
    xf = x.astype(F32)
    t128 = t.astype(F32)
    w5 = jnp.pad(jnp.reshape(W.astype(F32), (5,)), (0, 11))
    bc = jnp.reshape(_pass1(xf, idx, t128, w5), (S, 2 * D))
    out = _pass2(xf, idx, w5, bc)
    return jnp.reshape(out, (S, D))


# speedup vs baseline: 6.2337x; 6.2337x over previous
import functools

import jax
import jax.numpy as jnp
from jax import lax
from jax.experimental import pallas as pl
from jax.experimental.pallas import tpu as pltpu
from jax.experimental.pallas import tpu_sc as plsc

N = 320000
D = 128
S = 10000
NW = 32
RPW = N // NW
CH = 80
NCH = RPW // CH
NG = CH // 16
NB_PAD = 96
NBATCH = NB_PAD // 16
RING = 8
F32 = jnp.float32
I32 = jnp.int32

_mesh = plsc.VectorSubcoreMesh(core_axis_name="c", subcore_axis_name="s")
_params = pltpu.CompilerParams(needs_layout_passes=False)


def _sread(ref, i):
    return ref[pl.ds(i, 16)][0]


def _scan_chunk(ib, base, prev0, pos, ids2=None):
    lanes = lax.iota(I32, 16)
    nb = jnp.int32(0)
    prev = prev0
    for g in range(NG):
        v = ib[pl.ds(base + g * 16, 16)]
        shifted = jnp.take(v, jnp.maximum(lanes - 1, 0))
        shifted = jnp.where(lanes == 0, prev, shifted)
        m = v != shifted
        cs = jnp.cumsum(m.astype(I32))
        tgt = nb + cs - 1
        plsc.store_scatter(pos, [tgt], lanes + g * 16, mask=m)
        if ids2 is not None:
            plsc.store_scatter(ids2, [tgt // 16, tgt % 16], v, mask=m)
        nb = nb + cs[15]
        prev = v[15]
    cur = pos[pl.ds(nb, 16)]
    pos[pl.ds(nb, 16)] = jnp.where(lanes == 0, CH, cur)
    if ids2 is not None:
        for b in range(NBATCH):
            lanepos = lanes + b * 16
            curi = ids2[b]
            ids2[b] = jnp.where(lanepos < nb, curi, 0)
    return nb


def _p1_body(x_hbm, idx_hbm, t_hbm, w_hbm, bc_hbm, ib, xb, pos, prevb, tb,
             wb, ring, xsem, fsem):
    wid = lax.axis_index("s") * 2 + lax.axis_index("c")
    r0 = pl.multiple_of(wid * RPW, 8)

    pltpu.sync_copy(idx_hbm.at[pl.ds(r0, RPW)], ib.at[pl.ds(0, RPW)])
    pltpu.sync_copy(t_hbm, tb)
    pltpu.sync_copy(w_hbm, wb)
    for j in range(8):
        tv = tb[pl.ds(j * 16, 16)]
        tb[pl.ds(j * 16, 16)] = jnp.clip(tv, 0.0, 1.0)

    @pl.when(wid > 0)
    def _():
        pltpu.sync_copy(idx_hbm.at[pl.ds(pl.multiple_of(r0 - 8, 8), 16)], prevb)

    tile_prev = jnp.where(wid > 0, prevb[pl.ds(0, 16)][7], jnp.int32(-1))
    wv = wb[pl.ds(0, 16)]
    w0, w1, w2, w4 = wv[0], wv[1], wv[2], wv[4]

    pinf = jnp.full((16,), 3.4e38, F32)
    ninf = jnp.full((16,), -3.4e38, F32)
    zero = jnp.zeros((16,), F32)

    def flush(cur_seg, cnt, mn, mx, sm, dma_cnt):
        @pl.when(cur_seg != tile_prev)
        def _():
            @pl.when(dma_cnt >= RING)
            def _():
                pltpu.make_async_copy(ring.at[0], bc_hbm.at[pl.ds(0, 2 * D)],
                                      fsem).wait()

            slot = dma_cnt & (RING - 1)
            cntf = cnt.astype(F32)
            for j in range(8):
                ttj = tb[pl.ds(j * 16, 16)]
                ring[slot, pl.ds(j * 16, 16)] = (
                    ttj * mx[j] + (1.0 - ttj) * mn[j])
                ring[slot, pl.ds(128 + j * 16, 16)] = (
                    w0 * cntf + w1 * mn[j] + w2 * mx[j] + w4 * sm[j])
            pltpu.async_copy(
                ring.at[slot],
                bc_hbm.at[pl.ds(pl.multiple_of(cur_seg * (2 * D), 8),
                               2 * D)], fsem)

        return jnp.where(cur_seg != tile_prev, dma_cnt + 1, dma_cnt)

    def accum_rows(xbase, lo, hi, cnt, mn, mx, sm):
        def row(i, carry):
            cnt, mn, mx, sm = carry
            mn2, mx2, sm2 = [], [], []
            for j in range(8):
                ld = xb[xbase + i, pl.ds(j * 16, 16)]
                mn2.append(jnp.minimum(mn[j], ld))
                mx2.append(jnp.maximum(mx[j], ld))
                sm2.append(sm[j] + ld)
            return cnt + 1, tuple(mn2), tuple(mx2), tuple(sm2)

        return lax.fori_loop(lo, hi, row, (cnt, mn, mx, sm))

    pltpu.async_copy(x_hbm.at[pl.ds(r0, CH)], xb.at[pl.ds(0, CH)],
                     xsem)

    def chunk(k, carry):
        cur_seg, cnt, dma_cnt, mn, mx, sm = carry
        xbase = (k & 1) * CH
        pltpu.make_async_copy(x_hbm.at[pl.ds(0, CH)], xb.at[pl.ds(0, CH)],
                              xsem).wait()

        @pl.when(k + 1 < NCH)
        def _():
            pltpu.async_copy(x_hbm.at[pl.ds(pl.multiple_of(r0 + (k + 1) * CH, 8),
                                            CH)],
                             xb.at[pl.ds(((k + 1) & 1) * CH, CH)], xsem)

        prev0 = jnp.where(k > 0, _sread(ib, jnp.maximum(k * CH - 1, 0)),
                          tile_prev)
        nb = _scan_chunk(ib, k * CH, prev0, pos)
        p0 = jnp.where(nb > 0, _sread(pos, 0), CH)
        cnt, mn, mx, sm = accum_rows(xbase, 0, p0, cnt, mn, mx, sm)

        def seg(j, c):
            cur_seg, cnt, dma_cnt, mn, mx, sm = c
            p_lo = _sread(pos, j)
            p_hi = _sread(pos, j + 1)
            dma_cnt = flush(cur_seg, cnt, mn, mx, sm, dma_cnt)
            new_seg = _sread(ib, k * CH + p_lo)
            cnt, mn, mx, sm = accum_rows(
                xbase, p_lo, p_hi, jnp.int32(0),
                (pinf,) * 8, (ninf,) * 8, (zero,) * 8)
            return new_seg, cnt, dma_cnt, mn, mx, sm

        return lax.fori_loop(0, nb, seg,
                             (cur_seg, cnt, dma_cnt, mn, mx, sm))

    init = (tile_prev, jnp.int32(0), jnp.int32(0),
            (pinf,) * 8, (ninf,) * 8, (zero,) * 8)
    cur_seg, cnt, dma_cnt, mn, mx, sm = lax.fori_loop(0, NCH, chunk, init)

    first_ec = pl.multiple_of(r0 + RPW, 8)

    @pl.when(first_ec < N)
    def _():
        pltpu.sync_copy(idx_hbm.at[pl.ds(pl.multiple_of(first_ec, 8), 16)], prevb)

    nxt_id = jnp.where(first_ec < N, prevb[pl.ds(0, 16)][0], jnp.int32(-1))
    cont0 = (nxt_id == cur_seg) & (first_ec < N)

    def ext_cond(c):
        return c[0]

    def ext_body(c):
        _, ec0, cnt, mn, mx, sm = c
        pltpu.sync_copy(idx_hbm.at[pl.ds(pl.multiple_of(ec0, 8), CH)],
                        ib.at[pl.ds(0, CH)])
        pltpu.sync_copy(x_hbm.at[pl.ds(pl.multiple_of(ec0, 8), CH)],
                        xb.at[pl.ds(0, CH)])
        nb = _scan_chunk(ib, 0, cur_seg, pos)
        fp = jnp.where(nb > 0, _sread(pos, 0), CH)
        cnt, mn, mx, sm = accum_rows(0, 0, fp, cnt, mn, mx, sm)
        cont = (nb == 0) & (ec0 + CH < N)
        return cont, pl.multiple_of(ec0 + CH, 8), cnt, mn, mx, sm

    _, _, cnt, mn, mx, sm = lax.while_loop(
        ext_cond, ext_body, (cont0, first_ec, cnt, mn, mx, sm))

    dma_cnt = flush(cur_seg, cnt, mn, mx, sm, dma_cnt)

    def drain(i, _):
        @pl.when(i < jnp.minimum(dma_cnt, RING))
        def _():
            pltpu.make_async_copy(ring.at[0], bc_hbm.at[pl.ds(0, 2 * D)],
                                      fsem).wait()

        return 0

    lax.fori_loop(0, RING, drain, 0)


def _p2_body(x_hbm, idx_hbm, w_hbm, bc_hbm, out_hbm, ib, xb, pos2, ids2,
             slab, prevb, wb, ring, zbuf, xsem, gsem, fsem):
    wid = lax.axis_index("s") * 2 + lax.axis_index("c")
    r0 = pl.multiple_of(wid * RPW, 8)

    pltpu.sync_copy(idx_hbm.at[pl.ds(r0, RPW)], ib.at[pl.ds(0, RPW)])
    pltpu.sync_copy(w_hbm, wb)

    @pl.when(wid > 0)
    def _():
        pltpu.sync_copy(idx_hbm.at[pl.ds(pl.multiple_of(r0 - 8, 8), 16)], prevb)

    tile_prev = jnp.where(wid > 0, prevb[pl.ds(0, 16)][7], jnp.int32(-1))
    w3 = wb[pl.ds(0, 16)][3]
    zero = jnp.zeros((16,), F32)
    for r in range(16 * 8):
        zbuf[pl.ds(r * 16, 16)] = zero

    def zfill(lo, hi):
        nfull = jnp.maximum((hi - lo) // 16, 0)

        def f16(i, _):
            pltpu.sync_copy(
                zbuf,
                out_hbm.at[pl.ds(pl.multiple_of((lo + i * 16) * D, 8),
                                 16 * D)])
            return 0

        lax.fori_loop(0, nfull, f16, 0)

        def f1(g, _):
            pltpu.sync_copy(
                zbuf.at[pl.ds(0, D)],
                out_hbm.at[pl.ds(pl.multiple_of(g * D, 8), D)])
            return 0

        lax.fori_loop(lo + nfull * 16, hi, f1, 0)

    def flush(cur_seg, new_seg, acc, cvec, dma_cnt):
        zfill(cur_seg + 1, new_seg)

        @pl.when(cur_seg != tile_prev)
        def _():
            @pl.when(dma_cnt >= RING)
            def _():
                pltpu.make_async_copy(ring.at[0],
                                      out_hbm.at[pl.ds(0, D)],
                                      fsem).wait()

            slot = dma_cnt & (RING - 1)
            for j in range(8):
                ring[slot, pl.ds(j * 16, 16)] = cvec[j] + w3 * acc[j]
            pltpu.async_copy(
                ring.at[slot],
                out_hbm.at[pl.ds(pl.multiple_of(cur_seg * D, 8), D)], fsem)

        return jnp.where(cur_seg != tile_prev, dma_cnt + 1, dma_cnt)

    def relu_rows(xbase, lo, hi, acc, bias):
        def row(i, a):
            a2 = []
            for j in range(8):
                ld = xb[xbase + i, pl.ds(j * 16, 16)]
                a2.append(a[j] + jnp.maximum(ld - bias[j], 0.0))
            return tuple(a2)

        return lax.fori_loop(lo, hi, row, acc)

    def scan_and_gather(kk, par):
        prev0 = _sread(ib, jnp.maximum(kk * CH - 1, 0))
        prev0 = jnp.where(kk > 0, prev0, tile_prev)
        nb = _scan_chunk(ib, kk * CH, prev0, pos2.at[par], ids2.at[par])
        for b in range(NBATCH):
            @pl.when(b * 16 < nb)
            def _():
                pltpu.async_copy(bc_hbm.at[ids2.at[par, b]],
                                 slab.at[pl.ds(par * NB_PAD + b * 16, 16)],
                                 gsem)
        return nb

    pltpu.async_copy(x_hbm.at[pl.ds(r0, CH)], xb.at[pl.ds(0, CH)],
                     xsem)
    nb0 = scan_and_gather(jnp.int32(0), 0)

    def chunk(k, carry):
        cur_seg, dma_cnt, nb, acc, bias, cvec = carry
        par = k & 1
        xbase = par * CH
        pltpu.make_async_copy(x_hbm.at[pl.ds(0, CH)], xb.at[pl.ds(0, CH)],
                              xsem).wait()
        for _b in range(NBATCH):
            @pl.when(_b * 16 < nb)
            def _():
                pltpu.make_async_copy(bc_hbm.at[ids2.at[0, 0]],
                                      slab.at[pl.ds(0, 16)], gsem).wait()

        @pl.when(k + 1 < NCH)
        def _():
            pltpu.async_copy(x_hbm.at[pl.ds(pl.multiple_of(r0 + (k + 1) * CH, 8),
                                            CH)],
                             xb.at[pl.ds(((k + 1) & 1) * CH, CH)], xsem)

        nb_next = scan_and_gather(jnp.minimum(k + 1, NCH - 1), (k + 1) & 1)

        p0 = jnp.where(nb > 0, _sread(pos2.at[par], 0), CH)
        acc = relu_rows(xbase, 0, p0, acc, bias)

        def seg(j, c):
            cur_seg, dma_cnt, acc, bias, cvec = c
            p_lo = _sread(pos2.at[par], j)
            p_hi = _sread(pos2.at[par], j + 1)
            new_seg = _sread(ib, k * CH + p_lo)
            dma_cnt = flush(cur_seg, new_seg, acc, cvec, dma_cnt)
            srow = par * NB_PAD + j
            nbias = tuple(
                slab[srow, pl.ds(jj * 16, 16)] for jj in range(8))
            ncvec = tuple(
                slab[srow, pl.ds(128 + jj * 16, 16)] for jj in range(8))
            acc = relu_rows(xbase, p_lo, p_hi, (zero,) * 8, nbias)
            return new_seg, dma_cnt, acc, nbias, ncvec

        cur_seg, dma_cnt, acc, bias, cvec = lax.fori_loop(
            0, nb, seg, (cur_seg, dma_cnt, acc, bias, cvec))
        return cur_seg, dma_cnt, nb_next, acc, bias, cvec

    init = (tile_prev, jnp.int32(0), nb0, (zero,) * 8, (zero,) * 8,
            (zero,) * 8)
    cur_seg, dma_cnt, _nb_end_scalar, acc, bias, cvec = lax.fori_loop(
        0, NCH, chunk, init)
    _nb_end = (_nb_end_scalar,)

    nb_last = cur_seg * 0 + _nb_end[0]
    for _b in range(NBATCH):
        @pl.when(_b * 16 < nb_last)
        def _():
            pltpu.make_async_copy(bc_hbm.at[ids2.at[0, 0]],
                                  slab.at[pl.ds(0, 16)], gsem).wait()

    first_ec = pl.multiple_of(r0 + RPW, 8)

    @pl.when(first_ec < N)
    def _():
        pltpu.sync_copy(idx_hbm.at[pl.ds(pl.multiple_of(first_ec, 8), 16)], prevb)

    nxt_id = jnp.where(first_ec < N, prevb[pl.ds(0, 16)][0], jnp.int32(-1))
    cont0 = (nxt_id == cur_seg) & (first_ec < N)

    def ext_cond(c):
        return c[0]

    def ext_body(c):
        _, ec0, acc = c
        pltpu.sync_copy(idx_hbm.at[pl.ds(pl.multiple_of(ec0, 8), CH)],
                        ib.at[pl.ds(0, CH)])
        pltpu.sync_copy(x_hbm.at[pl.ds(pl.multiple_of(ec0, 8), CH)],
                        xb.at[pl.ds(0, CH)])
        nb = _scan_chunk(ib, 0, cur_seg, pos2.at[0])
        fp = jnp.where(nb > 0, _sread(pos2.at[0], 0), CH)
        acc = relu_rows(0, 0, fp, acc, bias)
        cont = (nb == 0) & (ec0 + CH < N)
        return cont, pl.multiple_of(ec0 + CH, 8), acc

    _, _, acc = lax.while_loop(ext_cond, ext_body, (cont0, first_ec, acc))

    dma_cnt = flush(cur_seg, jnp.where(wid == NW - 1, S, cur_seg + 1),
                    acc, cvec, dma_cnt)

    def drain(i, _):
        @pl.when(i < jnp.minimum(dma_cnt, RING))
        def _():
            pltpu.make_async_copy(ring.at[0], out_hbm.at[pl.ds(0, D)],
                                      fsem).wait()

        return 0

    lax.fori_loop(0, RING, drain, 0)


_pass1 = functools.partial(
    pl.kernel,
    out_type=jax.ShapeDtypeStruct((S * 2 * D,), F32),
    mesh=_mesh,
    compiler_params=_params,
    scratch_types=[
        pltpu.VMEM((RPW + 16,), I32),
        pltpu.VMEM((2 * CH, D), F32),
        pltpu.VMEM((NB_PAD + 16,), I32),
        pltpu.VMEM((16,), I32),
        pltpu.VMEM((D,), F32),
        pltpu.VMEM((16,), F32),
        pltpu.VMEM((RING, 2 * D), F32),
        pltpu.SemaphoreType.DMA,
        pltpu.SemaphoreType.DMA,
    ],
)(_p1_body)

_pass2 = functools.partial(
    pl.kernel,
    out_type=jax.ShapeDtypeStruct((S * D,), F32),
    mesh=_mesh,
    compiler_params=_params,
    scratch_types=[
        pltpu.VMEM((RPW + 16,), I32),
        pltpu.VMEM((2 * CH, D), F32),
        pltpu.VMEM((2, NB_PAD + 16), I32),
        pltpu.VMEM((2, NBATCH, 16), I32),
        pltpu.VMEM((2 * NB_PAD, 2 * D), F32),
        pltpu.VMEM((16,), I32),
        pltpu.VMEM((16,), F32),
        pltpu.VMEM((RING, D), F32),
        pltpu.VMEM((16 * D,), F32),
        pltpu.SemaphoreType.DMA,
        pltpu.SemaphoreType.DMA,
        pltpu.SemaphoreType.DMA,
    ],
)(_p2_body)


def kernel(x, batch_idx, max_index, t, W):
    assert x.shape == (N, D)
    idx = batch_idx.astype(I32)
    xf = x.astype(F32)
    t128 = t.astype(F32)
    w5 = jnp.pad(jnp.reshape(W.astype(F32), (5,)), (0, 11))
    bc = jnp.reshape(_pass1(xf, idx, t128, w5), (S, 2 * D))
    out = _pass2(xf, idx, w5, bc)
    return jnp.reshape(out, (S, D))


# speedup vs baseline: 47.3202x; 7.5910x over previous
import functools

import jax
import jax.numpy as jnp
from jax import lax
from jax.experimental import pallas as pl
from jax.experimental.pallas import tpu as pltpu
from jax.experimental.pallas import tpu_sc as plsc

N = 320000
D = 128
S = 10000
NW = 32
RPW = N // NW
CH = 80
NCH = RPW // CH
NG = CH // 16
NB_PAD = 96
NBATCH = NB_PAD // 16
RING = 8
F32 = jnp.float32
I32 = jnp.int32

_mesh = plsc.VectorSubcoreMesh(core_axis_name="c", subcore_axis_name="s")
_params = pltpu.CompilerParams(needs_layout_passes=False)


def _sread(ref, i):
    return ref[pl.ds(i, 16)][0]


def _scan_chunk(ib, base, prev0, pos, ids2=None):
    lanes = lax.iota(I32, 16)
    nb = jnp.int32(0)
    prev = prev0
    for g in range(NG):
        v = ib[pl.ds(base + g * 16, 16)]
        shifted = jnp.take(v, jnp.maximum(lanes - 1, 0))
        shifted = jnp.where(lanes == 0, prev, shifted)
        m = v != shifted
        cs = jnp.cumsum(m.astype(I32))
        tgt = nb + cs - 1
        plsc.store_scatter(pos, [tgt], lanes + g * 16, mask=m)
        if ids2 is not None:
            plsc.store_scatter(ids2, [tgt // 16, tgt % 16], v, mask=m)
        nb = nb + cs[15]
        prev = v[15]
    cur = pos[pl.ds(nb, 16)]
    pos[pl.ds(nb, 16)] = jnp.where(lanes == 0, CH, cur)
    if ids2 is not None:
        for b in range(NBATCH):
            lanepos = lanes + b * 16
            curi = ids2[b]
            ids2[b] = jnp.where(lanepos < nb, curi, 0)
    return nb


def _p1_body(x_hbm, idx_hbm, t_hbm, w_hbm, bc_hbm, ib, xb, pos, prevb, tb,
             wb, ring, xsem, fsem):
    wid = lax.axis_index("s") * 2 + lax.axis_index("c")
    r0 = pl.multiple_of(wid * RPW, 8)

    pltpu.sync_copy(idx_hbm.at[pl.ds(r0, RPW)], ib.at[pl.ds(0, RPW)])
    pltpu.sync_copy(t_hbm, tb)
    pltpu.sync_copy(w_hbm, wb)
    for j in range(8):
        tv = tb[pl.ds(j * 16, 16)]
        tb[pl.ds(j * 16, 16)] = jnp.clip(tv, 0.0, 1.0)

    @pl.when(wid > 0)
    def _():
        pltpu.sync_copy(idx_hbm.at[pl.ds(pl.multiple_of(r0 - 8, 8), 16)], prevb)

    tile_prev = jnp.where(wid > 0, prevb[pl.ds(0, 16)][7], jnp.int32(-1))
    wv = wb[pl.ds(0, 16)]
    w0, w1, w2, w4 = wv[0], wv[1], wv[2], wv[4]

    pinf = jnp.full((16,), 3.4e38, F32)
    ninf = jnp.full((16,), -3.4e38, F32)
    zero = jnp.zeros((16,), F32)

    def flush(cur_seg, cnt, mn, mx, sm, dma_cnt):
        @pl.when(cur_seg != tile_prev)
        def _():
            @pl.when(dma_cnt >= RING)
            def _():
                pltpu.make_async_copy(ring.at[0], bc_hbm.at[pl.ds(0, 2 * D)],
                                      fsem).wait()

            slot = dma_cnt & (RING - 1)
            cntf = cnt.astype(F32)
            for j in range(8):
                ttj = tb[pl.ds(j * 16, 16)]
                ring[slot, pl.ds(j * 16, 16)] = (
                    ttj * mx[j] + (1.0 - ttj) * mn[j])
                ring[slot, pl.ds(128 + j * 16, 16)] = (
                    w0 * cntf + w1 * mn[j] + w2 * mx[j] + w4 * sm[j])
            pltpu.async_copy(
                ring.at[slot],
                bc_hbm.at[pl.ds(pl.multiple_of(cur_seg * (2 * D), 8),
                               2 * D)], fsem)

        return jnp.where(cur_seg != tile_prev, dma_cnt + 1, dma_cnt)

    def accum_rows(xbase, lo, hi, cnt, mn, mx, sm):
        def row(i, carry):
            cnt, mn, mx, sm = carry
            mn2, mx2, sm2 = [], [], []
            for j in range(8):
                ld = xb[xbase + i, pl.ds(j * 16, 16)]
                mn2.append(jnp.minimum(mn[j], ld))
                mx2.append(jnp.maximum(mx[j], ld))
                sm2.append(sm[j] + ld)
            return cnt + 1, tuple(mn2), tuple(mx2), tuple(sm2)

        return lax.fori_loop(lo, hi, row, (cnt, mn, mx, sm))

    pltpu.async_copy(x_hbm.at[pl.ds(r0, CH)], xb.at[pl.ds(0, CH)],
                     xsem)

    def chunk(k, carry):
        cur_seg, cnt, dma_cnt, mn, mx, sm = carry
        xbase = (k & 1) * CH
        pltpu.make_async_copy(x_hbm.at[pl.ds(0, CH)], xb.at[pl.ds(0, CH)],
                              xsem).wait()

        @pl.when(k + 1 < NCH)
        def _():
            pltpu.async_copy(x_hbm.at[pl.ds(pl.multiple_of(r0 + (k + 1) * CH, 8),
                                            CH)],
                             xb.at[pl.ds(((k + 1) & 1) * CH, CH)], xsem)

        prev0 = jnp.where(k > 0, _sread(ib, jnp.maximum(k * CH - 1, 0)),
                          tile_prev)
        nb = _scan_chunk(ib, k * CH, prev0, pos)
        p0 = jnp.where(nb > 0, _sread(pos, 0), CH)
        cnt, mn, mx, sm = accum_rows(xbase, 0, p0, cnt, mn, mx, sm)

        def seg(j, c):
            cur_seg, cnt, dma_cnt, mn, mx, sm = c
            p_lo = _sread(pos, j)
            p_hi = _sread(pos, j + 1)
            dma_cnt = flush(cur_seg, cnt, mn, mx, sm, dma_cnt)
            new_seg = _sread(ib, k * CH + p_lo)
            cnt, mn, mx, sm = accum_rows(
                xbase, p_lo, p_hi, jnp.int32(0),
                (pinf,) * 8, (ninf,) * 8, (zero,) * 8)
            return new_seg, cnt, dma_cnt, mn, mx, sm

        return lax.fori_loop(0, nb, seg,
                             (cur_seg, cnt, dma_cnt, mn, mx, sm))

    init = (tile_prev, jnp.int32(0), jnp.int32(0),
            (pinf,) * 8, (ninf,) * 8, (zero,) * 8)
    cur_seg, cnt, dma_cnt, mn, mx, sm = lax.fori_loop(0, NCH, chunk, init)

    first_ec = pl.multiple_of(r0 + RPW, 8)

    @pl.when(first_ec < N)
    def _():
        pltpu.sync_copy(idx_hbm.at[pl.ds(pl.multiple_of(first_ec, 8), 16)], prevb)

    nxt_id = jnp.where(first_ec < N, prevb[pl.ds(0, 16)][0], jnp.int32(-1))
    cont0 = (nxt_id == cur_seg) & (first_ec < N)

    def ext_cond(c):
        return c[0]

    def ext_body(c):
        _, ec0, cnt, mn, mx, sm = c
        pltpu.sync_copy(idx_hbm.at[pl.ds(pl.multiple_of(ec0, 8), CH)],
                        ib.at[pl.ds(0, CH)])
        pltpu.sync_copy(x_hbm.at[pl.ds(pl.multiple_of(ec0, 8), CH)],
                        xb.at[pl.ds(0, CH)])
        nb = _scan_chunk(ib, 0, cur_seg, pos)
        fp = jnp.where(nb > 0, _sread(pos, 0), CH)
        cnt, mn, mx, sm = accum_rows(0, 0, fp, cnt, mn, mx, sm)
        cont = (nb == 0) & (ec0 + CH < N)
        return cont, pl.multiple_of(ec0 + CH, 8), cnt, mn, mx, sm

    _, _, cnt, mn, mx, sm = lax.while_loop(
        ext_cond, ext_body, (cont0, first_ec, cnt, mn, mx, sm))

    dma_cnt = flush(cur_seg, cnt, mn, mx, sm, dma_cnt)

    def drain(i, _):
        @pl.when(i < jnp.minimum(dma_cnt, RING))
        def _():
            pltpu.make_async_copy(ring.at[0], bc_hbm.at[pl.ds(0, 2 * D)],
                                      fsem).wait()

        return 0

    lax.fori_loop(0, RING, drain, 0)


def _p2_body(x_hbm, idx_hbm, w_hbm, bc_hbm, out_hbm, ib, xb, pos2,
             slab, prevb, wb, ring, zbuf, xsem, gsem, fsem):
    wid = lax.axis_index("s") * 2 + lax.axis_index("c")
    r0 = pl.multiple_of(wid * RPW, 8)

    pltpu.sync_copy(idx_hbm.at[pl.ds(r0, RPW)], ib.at[pl.ds(0, RPW)])
    pltpu.sync_copy(w_hbm, wb)

    @pl.when(wid > 0)
    def _():
        pltpu.sync_copy(idx_hbm.at[pl.ds(pl.multiple_of(r0 - 8, 8), 16)], prevb)

    tile_prev = jnp.where(wid > 0, prevb[pl.ds(0, 16)][7], jnp.int32(-1))
    w3 = wb[pl.ds(0, 16)][3]
    zero = jnp.zeros((16,), F32)
    for r in range(16 * 8):
        zbuf[pl.ds(r * 16, 16)] = zero

    def zfill(lo, hi):
        nfull = jnp.maximum((hi - lo) // 16, 0)

        def f16(i, _):
            pltpu.sync_copy(
                zbuf,
                out_hbm.at[pl.ds(pl.multiple_of((lo + i * 16) * D, 8),
                                 16 * D)])
            return 0

        lax.fori_loop(0, nfull, f16, 0)

        def f1(g, _):
            pltpu.sync_copy(
                zbuf.at[pl.ds(0, D)],
                out_hbm.at[pl.ds(pl.multiple_of(g * D, 8), D)])
            return 0

        lax.fori_loop(lo + nfull * 16, hi, f1, 0)

    def flush(cur_seg, new_seg, acc, cvec, dma_cnt):
        zfill(cur_seg + 1, new_seg)

        @pl.when(cur_seg != tile_prev)
        def _():
            @pl.when(dma_cnt >= RING)
            def _():
                pltpu.make_async_copy(ring.at[0],
                                      out_hbm.at[pl.ds(0, D)],
                                      fsem).wait()

            slot = dma_cnt & (RING - 1)
            for j in range(8):
                ring[slot, pl.ds(j * 16, 16)] = cvec[j] + w3 * acc[j]
            pltpu.async_copy(
                ring.at[slot],
                out_hbm.at[pl.ds(pl.multiple_of(cur_seg * D, 8), D)], fsem)

        return jnp.where(cur_seg != tile_prev, dma_cnt + 1, dma_cnt)

    def relu_rows(xbase, lo, hi, acc, bias):
        def row(i, a):
            a2 = []
            for j in range(8):
                ld = xb[xbase + i, pl.ds(j * 16, 16)]
                a2.append(a[j] + jnp.maximum(ld - bias[j], 0.0))
            return tuple(a2)

        return lax.fori_loop(lo, hi, row, acc)

    def scan_and_prefetch(kk, par):
        prev0 = _sread(ib, jnp.maximum(kk * CH - 1, 0))
        prev0 = jnp.where(kk > 0, prev0, tile_prev)
        nb = _scan_chunk(ib, kk * CH, prev0, pos2.at[par])

        def issue(j, _):
            sid = _sread(ib, kk * CH + _sread(pos2.at[par], j))
            pltpu.async_copy(
                bc_hbm.at[pl.ds(pl.multiple_of(sid * (2 * D), 8), 2 * D)],
                slab.at[par * NB_PAD + j], gsem)
            return 0

        lax.fori_loop(0, nb, issue, 0)
        return nb

    pltpu.async_copy(x_hbm.at[pl.ds(r0, CH)], xb.at[pl.ds(0, CH)],
                     xsem)
    nb0 = scan_and_prefetch(jnp.int32(0), 0)

    def chunk(k, carry):
        cur_seg, dma_cnt, nb, acc, bias, cvec = carry
        par = k & 1
        xbase = par * CH
        pltpu.make_async_copy(x_hbm.at[pl.ds(0, CH)], xb.at[pl.ds(0, CH)],
                              xsem).wait()
        def gwait(_j, _):
            pltpu.make_async_copy(bc_hbm.at[pl.ds(0, 2 * D)], slab.at[0],
                                  gsem).wait()
            return 0

        lax.fori_loop(0, nb, gwait, 0)

        @pl.when(k + 1 < NCH)
        def _():
            pltpu.async_copy(x_hbm.at[pl.ds(pl.multiple_of(r0 + (k + 1) * CH, 8),
                                            CH)],
                             xb.at[pl.ds(((k + 1) & 1) * CH, CH)], xsem)

        nb_next = scan_and_prefetch(jnp.minimum(k + 1, NCH - 1),
                                    (k + 1) & 1)

        p0 = jnp.where(nb > 0, _sread(pos2.at[par], 0), CH)
        acc = relu_rows(xbase, 0, p0, acc, bias)

        def seg(j, c):
            cur_seg, dma_cnt, acc, bias, cvec = c
            p_lo = _sread(pos2.at[par], j)
            p_hi = _sread(pos2.at[par], j + 1)
            new_seg = _sread(ib, k * CH + p_lo)
            dma_cnt = flush(cur_seg, new_seg, acc, cvec, dma_cnt)
            srow = par * NB_PAD + j
            nbias = tuple(
                slab[srow, pl.ds(jj * 16, 16)] for jj in range(8))
            ncvec = tuple(
                slab[srow, pl.ds(128 + jj * 16, 16)] for jj in range(8))
            acc = relu_rows(xbase, p_lo, p_hi, (zero,) * 8, nbias)
            return new_seg, dma_cnt, acc, nbias, ncvec

        cur_seg, dma_cnt, acc, bias, cvec = lax.fori_loop(
            0, nb, seg, (cur_seg, dma_cnt, acc, bias, cvec))
        return cur_seg, dma_cnt, nb_next, acc, bias, cvec

    init = (tile_prev, jnp.int32(0), nb0, (zero,) * 8, (zero,) * 8,
            (zero,) * 8)
    cur_seg, dma_cnt, _nb_end_scalar, acc, bias, cvec = lax.fori_loop(
        0, NCH, chunk, init)
    _nb_end = (_nb_end_scalar,)

    def gdrain(_j, _):
        pltpu.make_async_copy(bc_hbm.at[pl.ds(0, 2 * D)], slab.at[0],
                              gsem).wait()
        return 0

    lax.fori_loop(0, _nb_end[0], gdrain, 0)

    first_ec = pl.multiple_of(r0 + RPW, 8)

    @pl.when(first_ec < N)
    def _():
        pltpu.sync_copy(idx_hbm.at[pl.ds(pl.multiple_of(first_ec, 8), 16)], prevb)

    nxt_id = jnp.where(first_ec < N, prevb[pl.ds(0, 16)][0], jnp.int32(-1))
    cont0 = (nxt_id == cur_seg) & (first_ec < N)

    def ext_cond(c):
        return c[0]

    def ext_body(c):
        _, ec0, acc = c
        pltpu.sync_copy(idx_hbm.at[pl.ds(pl.multiple_of(ec0, 8), CH)],
                        ib.at[pl.ds(0, CH)])
        pltpu.sync_copy(x_hbm.at[pl.ds(pl.multiple_of(ec0, 8), CH)],
                        xb.at[pl.ds(0, CH)])
        nb = _scan_chunk(ib, 0, cur_seg, pos2.at[0])
        fp = jnp.where(nb > 0, _sread(pos2.at[0], 0), CH)
        acc = relu_rows(0, 0, fp, acc, bias)
        cont = (nb == 0) & (ec0 + CH < N)
        return cont, pl.multiple_of(ec0 + CH, 8), acc

    _, _, acc = lax.while_loop(ext_cond, ext_body, (cont0, first_ec, acc))

    dma_cnt = flush(cur_seg, jnp.where(wid == NW - 1, S, cur_seg + 1),
                    acc, cvec, dma_cnt)

    def drain(i, _):
        @pl.when(i < jnp.minimum(dma_cnt, RING))
        def _():
            pltpu.make_async_copy(ring.at[0], out_hbm.at[pl.ds(0, D)],
                                      fsem).wait()

        return 0

    lax.fori_loop(0, RING, drain, 0)


_pass1 = functools.partial(
    pl.kernel,
    out_type=jax.ShapeDtypeStruct((S * 2 * D,), F32),
    mesh=_mesh,
    compiler_params=_params,
    scratch_types=[
        pltpu.VMEM((RPW + 16,), I32),
        pltpu.VMEM((2 * CH, D), F32),
        pltpu.VMEM((NB_PAD + 16,), I32),
        pltpu.VMEM((16,), I32),
        pltpu.VMEM((D,), F32),
        pltpu.VMEM((16,), F32),
        pltpu.VMEM((RING, 2 * D), F32),
        pltpu.SemaphoreType.DMA,
        pltpu.SemaphoreType.DMA,
    ],
)(_p1_body)

_pass2 = functools.partial(
    pl.kernel,
    out_type=jax.ShapeDtypeStruct((S * D,), F32),
    mesh=_mesh,
    compiler_params=_params,
    scratch_types=[
        pltpu.VMEM((RPW + 16,), I32),
        pltpu.VMEM((2 * CH, D), F32),
        pltpu.VMEM((2, NB_PAD + 16), I32),
        pltpu.VMEM((2 * NB_PAD, 2 * D), F32),
        pltpu.VMEM((16,), I32),
        pltpu.VMEM((16,), F32),
        pltpu.VMEM((RING, D), F32),
        pltpu.VMEM((16 * D,), F32),
        pltpu.SemaphoreType.DMA,
        pltpu.SemaphoreType.DMA,
        pltpu.SemaphoreType.DMA,
    ],
)(_p2_body)


def kernel(x, batch_idx, max_index, t, W):
    assert x.shape == (N, D)
    idx = batch_idx.astype(I32)
    xf = x.astype(F32)
    t128 = t.astype(F32)
    w5 = jnp.pad(jnp.reshape(W.astype(F32), (5,)), (0, 11))
    bc = _pass1(xf, idx, t128, w5)
    out = _pass2(xf, idx, w5, bc)
    return jnp.reshape(out, (S, D))
